# baseline (speedup 1.0000x reference)
import functools

import jax
import jax.numpy as jnp
from jax import lax
from jax.experimental import pallas as pl
from jax.experimental.pallas import tpu as pltpu
from jax.experimental.pallas import tpu_sc as plsc

NC = 2
NS = 16
NW = NC * NS
LANES = 16
CHUNK = 128


def _matmul_body(x_ref, w_ref, o_ref):
    o_ref[...] = jnp.dot(x_ref[...], w_ref[...],
                         preferred_element_type=jnp.float32)


def _combine_body(p0_ref, p1_ref, b_ref, o_ref):
    o_ref[...] = p0_ref[...] + p1_ref[...] + b_ref[...]


def _make_spmm(n_nodes, d, nck):
    zpt = (((n_nodes + NS - 1) // NS) + 7) // 8 * 8
    spm_rows = zpt * NS
    opt = (n_nodes // NS) // 8 * 8
    o_tail = n_nodes - opt * NS
    h_full = n_nodes // zpt
    h_tail = n_nodes - h_full * zpt
    assert n_nodes % 8 == 0 and o_tail < CHUNK and h_tail % 8 == 0
    npairs = nck // 2
    nckh = nck // 2
    assert nck % 4 == 0
    p_src = npairs // 2 - 1
    p_dw = npairs // 2

    mesh = plsc.VectorSubcoreMesh(core_axis_name="c", subcore_axis_name="s")

    @functools.partial(
        pl.kernel,
        out_type=[
            jax.ShapeDtypeStruct((NC * n_nodes, d), jnp.float32),
            jax.ShapeDtypeStruct((NW * nck * CHUNK, d), jnp.float32),
        ],
        mesh=mesh,
        scratch_types=[
            pltpu.VMEM((nckh, CHUNK), jnp.int32),
            pltpu.VMEM((nckh, CHUNK), jnp.float32),
            pltpu.VMEM((CHUNK, d), jnp.float32),
            pltpu.VMEM((CHUNK, d), jnp.float32),
            pltpu.VMEM_SHARED((spm_rows, d), jnp.float32),
            pltpu.SemaphoreType.DMA,
            pltpu.SemaphoreType.DMA,
            pltpu.SemaphoreType.DMA,
            pltpu.SemaphoreType.DMA,
        ],
    )
    def spmm(h_hbm, src_hbm, dst_hbm, w_hbm, out_hbm, msgs_hbm,
             idx_v, w_v, rows0_v, rows1_v, spm,
             gsem0, gsem1, ssem0, ssem1):
        cid = lax.axis_index("c")
        sid = lax.axis_index("s")
        wid = sid * NC + cid
        zero = jnp.zeros((LANES,), jnp.float32)
        rows = (rows0_v, rows1_v)
        gsem = (gsem0, gsem1)
        ssem = (ssem0, ssem1)
        mrow0 = wid * nck * CHUNK

        pltpu.sync_copy(src_hbm.at[wid, 0], idx_v)
        pltpu.sync_copy(w_hbm.at[wid, 0], w_v)

        hb0 = sid * zpt

        @pl.when(sid < h_full)
        def _():
            pltpu.sync_copy(h_hbm.at[pl.ds(hb0, zpt)],
                            spm.at[pl.ds(hb0, zpt)])
        if h_tail:
            @pl.when(sid == h_full)
            def _():
                pltpu.sync_copy(h_hbm.at[pl.ds(hb0, h_tail)],
                                spm.at[pl.ds(hb0, h_tail)])
        plsc.subcore_barrier()

        def scale_buf(buf, ck):
            def scale(g, carry):
                wg = w_v[ck, pl.ds(g * LANES, LANES)]
                for l in range(LANES):
                    wvec = lax.gather(
                        wg, jnp.full((LANES, 1), l, jnp.int32),
                        lax.GatherDimensionNumbers(
                            offset_dims=(), collapsed_slice_dims=(0,),
                            start_index_map=(0,)),
                        slice_sizes=(1,),
                        mode=lax.GatherScatterMode.PROMISE_IN_BOUNDS)
                    e = g * LANES + l
                    for c2 in range(d // LANES):
                        sl = pl.ds(c2 * LANES, LANES)
                        buf[e, sl] = buf[e, sl] * wvec
                return carry
            lax.fori_loop(0, CHUNK // LANES, scale, 0)

        pltpu.async_copy(spm.at[idx_v.at[0]], rows0_v, gsem0)

        def p1_body(p, carry):
            for b in (0, 1):
                ci = 2 * p + b
                ck = lax.rem(ci, nckh)
                cn = lax.rem(ci + 1, nckh)
                pltpu.make_async_copy(
                    spm.at[idx_v.at[ck]], rows[b], gsem[b]).wait()
                if b == 0:
                    @pl.when(p > 0)
                    def _():
                        pltpu.make_async_copy(
                            rows[1], msgs_hbm.at[pl.ds(mrow0, CHUNK)],
                            ssem[1]).wait()

                    @pl.when(p == p_dw)
                    def _():
                        pltpu.sync_copy(w_hbm.at[wid, 1], w_v)
                    pltpu.async_copy(
                        spm.at[idx_v.at[cn]], rows[1], gsem[1])
                else:
                    pltpu.make_async_copy(
                        rows[0], msgs_hbm.at[pl.ds(mrow0, CHUNK)],
                        ssem[0]).wait()

                    @pl.when(p == p_src)
                    def _():
                        pltpu.sync_copy(src_hbm.at[wid, 1], idx_v)

                    @pl.when(p < npairs - 1)
                    def _():
                        pltpu.async_copy(
                            spm.at[idx_v.at[cn]], rows[0], gsem[0])
                scale_buf(rows[b], ck)
                pltpu.async_copy(
                    rows[b], msgs_hbm.at[pl.ds(mrow0 + ci * CHUNK, CHUNK)],
                    ssem[b])
            return carry
        lax.fori_loop(0, npairs, p1_body, 0)
        pltpu.make_async_copy(
            rows1_v, msgs_hbm.at[pl.ds(mrow0, CHUNK)], ssem[1]).wait()
        plsc.subcore_barrier()

        def zrow(i, carry):
            for g in range(d // LANES):
                rows0_v[i, pl.ds(g * LANES, LANES)] = zero
            return carry
        lax.fori_loop(0, CHUNK, zrow, 0)

        n_full = zpt // CHUNK
        rem = zpt - n_full * CHUNK
        for k in range(n_full):
            pltpu.sync_copy(rows0_v, spm.at[pl.ds(hb0 + k * CHUNK, CHUNK)])
        if rem:
            pltpu.sync_copy(rows0_v.at[pl.ds(0, rem)],
                            spm.at[pl.ds(hb0 + n_full * CHUNK, rem)])
        pltpu.sync_copy(dst_hbm.at[wid, 0], idx_v)
        plsc.subcore_barrier()

        pltpu.async_copy(msgs_hbm.at[pl.ds(mrow0, CHUNK)], rows0_v, gsem0)

        def p2_body(p, carry):
            for b in (0, 1):
                ci = 2 * p + b
                ck = lax.rem(ci, nckh)
                cn = lax.rem(ci + 1, nckh)
                pltpu.make_async_copy(
                    msgs_hbm.at[pl.ds(mrow0, CHUNK)], rows[b],
                    gsem[b]).wait()
                if b == 0:
                    @pl.when(p > 0)
                    def _():
                        pltpu.make_async_copy(
                            rows[1], spm.at[idx_v.at[ck]], ssem[1]).wait()

                    @pl.when(p == p_dw)
                    def _():
                        pltpu.sync_copy(dst_hbm.at[wid, 1], idx_v)
                    pltpu.async_copy(
                        msgs_hbm.at[pl.ds(mrow0 + (ci + 1) * CHUNK, CHUNK)],
                        rows[1], gsem[1])
                else:
                    pltpu.make_async_copy(
                        rows[0], spm.at[idx_v.at[ck]], ssem[0]).wait()

                    @pl.when(p < npairs - 1)
                    def _():
                        pltpu.async_copy(
                            msgs_hbm.at[
                                pl.ds(mrow0 + (ci + 1) * CHUNK, CHUNK)],
                            rows[0], gsem[0])
                pltpu.async_copy(
                    rows[b], spm.at[idx_v.at[ck]], ssem[b], add=True)
            return carry
        lax.fori_loop(0, npairs, p2_body, 0)
        pltpu.make_async_copy(
            rows1_v, spm.at[idx_v.at[0]], ssem[1]).wait()
        plsc.subcore_barrier()

        obase = sid * opt
        hbase = cid * n_nodes + obase
        o_full = opt // CHUNK
        orem = opt - o_full * CHUNK
        for k in range(o_full):
            pltpu.sync_copy(spm.at[pl.ds(obase + k * CHUNK, CHUNK)], rows0_v)
            pltpu.sync_copy(rows0_v,
                            out_hbm.at[pl.ds(hbase + k * CHUNK, CHUNK)])
        if orem:
            r0 = o_full * CHUNK
            pltpu.sync_copy(spm.at[pl.ds(obase + r0, orem)],
                            rows0_v.at[pl.ds(0, orem)])
            pltpu.sync_copy(rows0_v.at[pl.ds(0, orem)],
                            out_hbm.at[pl.ds(hbase + r0, orem)])
        if o_tail:
            @pl.when(sid == 0)
            def _():
                t0 = opt * NS
                pltpu.sync_copy(spm.at[pl.ds(t0, o_tail)],
                                rows0_v.at[pl.ds(0, o_tail)])
                pltpu.sync_copy(rows0_v.at[pl.ds(0, o_tail)],
                                out_hbm.at[pl.ds(cid * n_nodes + t0, o_tail)])

    return spmm


def kernel(x, edge_index, edge_weight, W, b):
    n, d_in = x.shape
    d_out = W.shape[1]
    e = edge_weight.shape[0]

    nck = -(-e // (NW * CHUNK))
    nck = ((nck + 15) // 16) * 16
    epad = nck * CHUNK * NW
    pad = epad - e
    src = jnp.pad(edge_index[0].astype(jnp.int32),
                  (0, pad)).reshape(NW, 2, nck // 2, CHUNK)
    dst = jnp.pad(edge_index[1].astype(jnp.int32),
                  (0, pad)).reshape(NW, 2, nck // 2, CHUNK)
    ew = jnp.pad(edge_weight, (0, pad)).reshape(NW, 2, nck // 2, CHUNK)

    blk = 1000
    h = pl.pallas_call(
        _matmul_body,
        grid=(n // blk,),
        in_specs=[pl.BlockSpec((blk, d_in), lambda i: (i, 0)),
                  pl.BlockSpec((d_in, d_out), lambda i: (0, 0))],
        out_specs=pl.BlockSpec((blk, d_out), lambda i: (i, 0)),
        out_shape=jax.ShapeDtypeStruct((n, d_out), jnp.float32),
    )(x, W)

    partials, _ = _make_spmm(n, d_out, nck)(h, src, dst, ew)

    b2 = b[None, :]
    nb = n // blk
    out = pl.pallas_call(
        _combine_body,
        grid=(nb,),
        in_specs=[pl.BlockSpec((blk, d_out), lambda i: (i, 0)),
                  pl.BlockSpec((blk, d_out), lambda i: (i + nb, 0)),
                  pl.BlockSpec((1, d_out), lambda i: (0, 0))],
        out_specs=pl.BlockSpec((blk, d_out), lambda i: (i, 0)),
        out_shape=jax.ShapeDtypeStruct((n, d_out), jnp.float32),
    )(partials, partials, b2)
    return out
